# baseline (speedup 1.0000x reference)
import jax
import jax.numpy as jnp
from jax.experimental import pallas as pl

_SDF = 128
_NPTS = 16
_LANES = _SDF * _NPTS


def _raster_body(yt_ref, xt_ref, res_ref, org_ref, out_ref):
    y = yt_ref[0]
    xx = xt_ref[0]
    res = res_ref[0]
    org = org_ref[0]
    row = jnp.clip((y / res[:, 0:1] + org[:, 0:1]).astype(jnp.int32), 0, _SDF - 1)
    col = jnp.clip((xx / res[:, 1:2] + org[:, 1:2]).astype(jnp.int32), 0, _SDF - 1)
    lane = jax.lax.broadcasted_iota(jnp.int32, (1, _LANES), 1)
    key = jnp.where(col == (lane >> 4), row, -1)
    key5 = key.reshape(2, 8, _SDF)
    ri = jax.lax.broadcasted_iota(jnp.int32, (_SDF, 2, 8, _SDF), 0)
    out_ref[0] = (ri == key5[None]).astype(jnp.float32)


def kernel(x, resolution, origin):
    b = x.shape[0]
    pts = x.reshape(b, _NPTS, 2)
    ys = jnp.tile(pts[:, :, 1], (1, _SDF)).reshape(b, 1, _LANES)
    xs = jnp.tile(pts[:, :, 0], (1, _SDF)).reshape(b, 1, _LANES)
    res3 = resolution.reshape(b, 1, 2)
    org3 = origin.reshape(b, 1, 2)
    out = pl.pallas_call(
        _raster_body,
        grid=(b,),
        in_specs=[
            pl.BlockSpec((1, 1, _LANES), lambda i: (i, 0, 0)),
            pl.BlockSpec((1, 1, _LANES), lambda i: (i, 0, 0)),
            pl.BlockSpec((1, 1, 2), lambda i: (i, 0, 0)),
            pl.BlockSpec((1, 1, 2), lambda i: (i, 0, 0)),
        ],
        out_specs=pl.BlockSpec((1, _SDF, 2, 8, _SDF), lambda i: (i, 0, 0, 0, 0)),
        out_shape=jax.ShapeDtypeStruct((b, _SDF, 2, 8, _SDF), jnp.float32),
    )(ys, xs, res3, org3)
    return out.reshape(b, _SDF, _SDF, _NPTS)


# speedup vs baseline: 3.2555x; 3.2555x over previous
import jax
import jax.numpy as jnp
from jax.experimental import pallas as pl

_SDF = 128
_NPTS = 16


def _raster_body(yt_ref, xt_ref, res_ref, org_ref, out_ref):
    yt = yt_ref[0]
    xt = xt_ref[0]
    res = res_ref[0]
    org = org_ref[0]
    row = jnp.clip((yt / res[:, 0:1] + org[:, 0:1]).astype(jnp.int32), 0, _SDF - 1)
    col = jnp.clip((xt / res[:, 1:2] + org[:, 1:2]).astype(jnp.int32), 0, _SDF - 1)
    ci = jax.lax.broadcasted_iota(jnp.int32, (_NPTS, _SDF), 1)
    key = jnp.where(col == ci, row, -1)
    ri = jax.lax.broadcasted_iota(jnp.int32, (_SDF, _NPTS, _SDF), 0)
    out_ref[0] = (ri == key[None]).astype(jnp.float32)


def kernel(x, resolution, origin):
    b = x.shape[0]
    pts = x.reshape(b, _NPTS, 2)
    ys = jnp.broadcast_to(pts[:, :, 1][:, :, None], (b, _NPTS, _SDF))
    xs = jnp.broadcast_to(pts[:, :, 0][:, :, None], (b, _NPTS, _SDF))
    res3 = resolution.reshape(b, 1, 2)
    org3 = origin.reshape(b, 1, 2)
    out = pl.pallas_call(
        _raster_body,
        grid=(b,),
        in_specs=[
            pl.BlockSpec((1, _NPTS, _SDF), lambda i: (i, 0, 0)),
            pl.BlockSpec((1, _NPTS, _SDF), lambda i: (i, 0, 0)),
            pl.BlockSpec((1, 1, 2), lambda i: (i, 0, 0)),
            pl.BlockSpec((1, 1, 2), lambda i: (i, 0, 0)),
        ],
        out_specs=pl.BlockSpec((1, _SDF, _NPTS, _SDF), lambda i: (i, 0, 0, 0)),
        out_shape=jax.ShapeDtypeStruct((b, _SDF, _NPTS, _SDF), jnp.float32),
    )(ys, xs, res3, org3)
    return jnp.transpose(out, (0, 1, 3, 2))


# speedup vs baseline: 4.8601x; 1.4929x over previous
import jax
import jax.numpy as jnp
from jax.experimental import pallas as pl
from jax.experimental.pallas import tpu as pltpu

_SDF = 128
_NPTS = 16
_BB = 2


def _raster_body(yt_ref, xt_ref, res_ref, org_ref, out_ref):
    for k in range(_BB):
        yt = yt_ref[k]
        xt = xt_ref[k]
        res = res_ref[k]
        org = org_ref[k]
        row = jnp.clip((yt / res[:, 0:1] + org[:, 0:1]).astype(jnp.int32), 0, _SDF - 1)
        col = jnp.clip((xt / res[:, 1:2] + org[:, 1:2]).astype(jnp.int32), 0, _SDF - 1)
        ci = jax.lax.broadcasted_iota(jnp.int32, (_NPTS, _SDF), 1)
        key = jnp.where(col == ci, row, -1)
        ri = jax.lax.broadcasted_iota(jnp.int32, (_SDF, _NPTS, _SDF), 0)
        out_ref[k] = (ri == key[None]).astype(jnp.float32)


def kernel(x, resolution, origin):
    b = x.shape[0]
    pts = x.reshape(b, _NPTS, 2)
    ys = jnp.broadcast_to(pts[:, :, 1][:, :, None], (b, _NPTS, _SDF))
    xs = jnp.broadcast_to(pts[:, :, 0][:, :, None], (b, _NPTS, _SDF))
    res3 = resolution.reshape(b, 1, 2)
    org3 = origin.reshape(b, 1, 2)
    out = pl.pallas_call(
        _raster_body,
        grid=(b // _BB,),
        in_specs=[
            pl.BlockSpec((_BB, _NPTS, _SDF), lambda i: (i, 0, 0)),
            pl.BlockSpec((_BB, _NPTS, _SDF), lambda i: (i, 0, 0)),
            pl.BlockSpec((_BB, 1, 2), lambda i: (i, 0, 0)),
            pl.BlockSpec((_BB, 1, 2), lambda i: (i, 0, 0)),
        ],
        out_specs=pl.BlockSpec((_BB, _SDF, _NPTS, _SDF), lambda i: (i, 0, 0, 0)),
        out_shape=jax.ShapeDtypeStruct((b, _SDF, _NPTS, _SDF), jnp.float32),
        compiler_params=pltpu.CompilerParams(
            dimension_semantics=("parallel",),
        ),
    )(ys, xs, res3, org3)
    return jnp.transpose(out, (0, 1, 3, 2))


# speedup vs baseline: 6.5374x; 1.3451x over previous
import jax
import jax.numpy as jnp
from jax.experimental import pallas as pl
from jax.experimental.pallas import tpu as pltpu

_SDF = 128
_NPTS = 16
_BB = 4


def _raster_body(yt_ref, xt_ref, res_ref, org_ref, out_ref):
    for k in range(_BB):
        yt = yt_ref[k]
        xt = xt_ref[k]
        res = res_ref[k]
        org = org_ref[k]
        row = jnp.clip((yt / res[:, 0:1] + org[:, 0:1]).astype(jnp.int32), 0, _SDF - 1)
        col = jnp.clip((xt / res[:, 1:2] + org[:, 1:2]).astype(jnp.int32), 0, _SDF - 1)
        ci = jax.lax.broadcasted_iota(jnp.int32, (_NPTS, _SDF), 1)
        key = jnp.where(col == ci, row, -1)
        ri = jax.lax.broadcasted_iota(jnp.int32, (_SDF, _NPTS, _SDF), 0)
        out_ref[k] = (ri == key[None]).astype(jnp.float32)


def kernel(x, resolution, origin):
    b = x.shape[0]
    pts = x.reshape(b, _NPTS, 2)
    ys = jnp.broadcast_to(pts[:, :, 1][:, :, None], (b, _NPTS, _SDF))
    xs = jnp.broadcast_to(pts[:, :, 0][:, :, None], (b, _NPTS, _SDF))
    res3 = resolution.reshape(b, 1, 2)
    org3 = origin.reshape(b, 1, 2)
    out = pl.pallas_call(
        _raster_body,
        grid=(b // _BB,),
        in_specs=[
            pl.BlockSpec((_BB, _NPTS, _SDF), lambda i: (i, 0, 0)),
            pl.BlockSpec((_BB, _NPTS, _SDF), lambda i: (i, 0, 0)),
            pl.BlockSpec((_BB, 1, 2), lambda i: (i, 0, 0)),
            pl.BlockSpec((_BB, 1, 2), lambda i: (i, 0, 0)),
        ],
        out_specs=pl.BlockSpec((_BB, _SDF, _NPTS, _SDF), lambda i: (i, 0, 0, 0)),
        out_shape=jax.ShapeDtypeStruct((b, _SDF, _NPTS, _SDF), jnp.float32),
        compiler_params=pltpu.CompilerParams(
            dimension_semantics=("parallel",),
        ),
    )(ys, xs, res3, org3)
    return jnp.transpose(out, (0, 1, 3, 2))


# speedup vs baseline: 6.8933x; 1.0544x over previous
import jax
import jax.numpy as jnp
from jax.experimental import pallas as pl
from jax.experimental.pallas import tpu as pltpu

_SDF = 128
_NPTS = 16
_BB = 8


def _raster_body(yt_ref, xt_ref, res_ref, org_ref, out_ref):
    for k in range(_BB):
        yt = yt_ref[k]
        xt = xt_ref[k]
        res = res_ref[k]
        org = org_ref[k]
        row = jnp.clip((yt / res[:, 0:1] + org[:, 0:1]).astype(jnp.int32), 0, _SDF - 1)
        col = jnp.clip((xt / res[:, 1:2] + org[:, 1:2]).astype(jnp.int32), 0, _SDF - 1)
        ci = jax.lax.broadcasted_iota(jnp.int32, (_NPTS, _SDF), 1)
        key = jnp.where(col == ci, row, -1)
        ri = jax.lax.broadcasted_iota(jnp.int32, (_SDF, _NPTS, _SDF), 0)
        out_ref[k] = (ri == key[None]).astype(jnp.float32)


def kernel(x, resolution, origin):
    b = x.shape[0]
    pts = x.reshape(b, _NPTS, 2)
    ys = jnp.broadcast_to(pts[:, :, 1][:, :, None], (b, _NPTS, _SDF))
    xs = jnp.broadcast_to(pts[:, :, 0][:, :, None], (b, _NPTS, _SDF))
    res3 = resolution.reshape(b, 1, 2)
    org3 = origin.reshape(b, 1, 2)
    out = pl.pallas_call(
        _raster_body,
        grid=(b // _BB,),
        in_specs=[
            pl.BlockSpec((_BB, _NPTS, _SDF), lambda i: (i, 0, 0)),
            pl.BlockSpec((_BB, _NPTS, _SDF), lambda i: (i, 0, 0)),
            pl.BlockSpec((_BB, 1, 2), lambda i: (i, 0, 0)),
            pl.BlockSpec((_BB, 1, 2), lambda i: (i, 0, 0)),
        ],
        out_specs=pl.BlockSpec((_BB, _SDF, _NPTS, _SDF), lambda i: (i, 0, 0, 0)),
        out_shape=jax.ShapeDtypeStruct((b, _SDF, _NPTS, _SDF), jnp.float32),
        compiler_params=pltpu.CompilerParams(
            dimension_semantics=("parallel",),
        ),
    )(ys, xs, res3, org3)
    return jnp.transpose(out, (0, 1, 3, 2))
